# baseline (speedup 1.0000x reference)
import jax
import jax.numpy as jnp
from jax import lax
from jax.experimental import pallas as pl
from jax.experimental.pallas import tpu as pltpu
from jax.experimental.pallas import tpu_sc as plsc

CO_IDX = 1000000
E = 16
B = 16384
F = 26
N = B * F
NC = 2
NS = 16
NW = NC * NS
BW = B // NW
CB = 128
JW = BW // CB
L = 16


NBUF = 6


def _co_embd_kernel(xi_hbm, xv_hbm, tab_hbm, out_hbm,
                    xiv, xvv, rows, stg,
                    gsem0, gsem1, gsem2, gsem3, gsem4, gsem5, wsem0, wsem1):
    gsems = (gsem0, gsem1, gsem2, gsem3, gsem4, gsem5)
    wid = lax.axis_index("s") * NC + lax.axis_index("c")
    b0 = wid * BW
    pltpu.sync_copy(xi_hbm.at[:, pl.ds(b0, BW)], xiv)
    pltpu.sync_copy(xv_hbm.at[:, pl.ds(b0, BW)], xvv)

    def fire(f, buf):
        for j in range(JW):
            pltpu.async_copy(
                tab_hbm.at[xiv.at[f, pl.ds(j * CB, CB)]],
                rows.at[buf, pl.ds(j * CB, CB)],
                gsems[buf],
            )

    def drain_gather(buf):
        pltpu.make_async_copy(
            tab_hbm.at[pl.ds(0, BW)], rows.at[buf],
            gsems[buf]).wait()

    iota = lax.iota(jnp.int32, L)
    for f0 in range(NBUF):
        fire(f0, f0)
    wcps = [None, None]
    for f in range(F):
        buf = f % NBUF
        if wcps[f % 2] is not None:
            wcps[f % 2].wait()
        drain_gather(buf)

        ob = f % 2

        @plsc.parallel_loop(0, BW // L)
        def col_group(t, f=f, buf=buf, ob=ob):
            cb = t * L
            xvec = xvv[f, pl.ds(cb, L)]
            row_idx = iota + cb
            bc = cb // CB
            bi = cb - bc * CB
            for er in range(2):
                for ei in range(8):
                    e = er * 8 + ei
                    g = plsc.load_gather(
                        rows.at[buf], [row_idx, jnp.full((L,), e, jnp.int32)])
                    stg[ob, er, bc, ei, pl.ds(bi, L)] = g * xvec
        if f + NBUF < F:
            fire(f + NBUF, buf)
        wcps[ob] = pltpu.async_copy(
            stg.at[ob], out_hbm.at[f, :, pl.ds(wid * JW, JW)],
            (wsem0, wsem1)[ob])
    for cp in wcps:
        if cp is not None:
            cp.wait()


@jax.jit
def kernel(xi, xv, co_emb_weight):
    xi_t = xi.T
    xv_t = xv.T
    mesh = plsc.VectorSubcoreMesh(core_axis_name="c", subcore_axis_name="s")
    out = pl.kernel(
        _co_embd_kernel,
        out_type=jax.ShapeDtypeStruct((F, 2, B // CB, 8, CB), jnp.float32),
        mesh=mesh,
        scratch_types=[
            pltpu.VMEM((F, BW), jnp.int32),
            pltpu.VMEM((F, BW), jnp.float32),
            pltpu.VMEM((NBUF, BW, E), jnp.float32),
            pltpu.VMEM((2, 2, JW, 8, CB), jnp.float32),
            pltpu.SemaphoreType.DMA,
            pltpu.SemaphoreType.DMA,
            pltpu.SemaphoreType.DMA,
            pltpu.SemaphoreType.DMA,
            pltpu.SemaphoreType.DMA,
            pltpu.SemaphoreType.DMA,
            pltpu.SemaphoreType.DMA,
            pltpu.SemaphoreType.DMA,
        ],
        compiler_params=pltpu.CompilerParams(
            use_tc_tiling_on_sc=False, needs_layout_passes=False),
    )(xi_t, xv_t, co_emb_weight)
    return out.transpose(2, 4, 0, 1, 3).reshape(B, F, E)
